# Initial kernel scaffold; baseline (speedup 1.0000x reference)
import jax
import jax.numpy as jnp
from jax.experimental import pallas as pl


def kernel(x, batch, W, b):
    raise NotImplementedError("write your pallas kernel here")


# speedup vs baseline: 13.0129x; 13.0129x over previous
import jax
import jax.numpy as jnp
from jax.experimental import pallas as pl
from jax.experimental.pallas import tpu as pltpu

_S = 512


def _block_fn(xb_ref, seg_ref, w_ref, b_ref, o_ref):
    i = pl.program_id(0)

    @pl.when(i == 0)
    def _init():
        o_ref[...] = jnp.zeros_like(o_ref)

    xb = xb_ref[...]
    z = jnp.sum(xb * w_ref[...], axis=1, keepdims=True) + b_ref[0, 0]
    wgt = jax.nn.sigmoid(z)
    y = wgt * xb
    seg = seg_ref[0]
    rows = jax.lax.broadcasted_iota(jnp.int32, (_S, seg.shape[-1]), 0)
    onehot = (rows == seg).astype(jnp.float32)
    o_ref[...] += jax.lax.dot(onehot, y, preferred_element_type=jnp.float32)


def kernel(x, batch, W, b):
    N, D = x.shape
    B = 2000
    G = N // B
    batch3 = batch.astype(jnp.int32).reshape(G, 1, B)
    Wr = W.reshape(1, D).astype(jnp.float32)
    br = b.reshape(1, 1).astype(jnp.float32)
    out = pl.pallas_call(
        _block_fn,
        grid=(G,),
        in_specs=[
            pl.BlockSpec((B, D), lambda i: (i, 0)),
            pl.BlockSpec((1, 1, B), lambda i: (i, 0, 0)),
            pl.BlockSpec((1, D), lambda i: (0, 0)),
            pl.BlockSpec((1, 1), lambda i: (0, 0)),
        ],
        out_specs=pl.BlockSpec((_S, D), lambda i: (0, 0)),
        out_shape=jax.ShapeDtypeStruct((_S, D), jnp.float32),
        compiler_params=pltpu.CompilerParams(
            dimension_semantics=("arbitrary",),
        ),
    )(x, batch3, Wr, br)
    return out
